# Initial kernel scaffold; baseline (speedup 1.0000x reference)
import jax
import jax.numpy as jnp
from jax.experimental import pallas as pl


def kernel(x0, x1, x2, edge_index0, edge_index1, edge_index2, W0, b0, W1, b1, W2, b2):
    raise NotImplementedError("write your pallas kernel here")


# speedup vs baseline: 12.1308x; 12.1308x over previous
import functools

import jax
import jax.numpy as jnp
from jax import lax
from jax.experimental import pallas as pl
from jax.experimental.pallas import tpu as pltpu
from jax.experimental.pallas import tpu_sc as plsc

NC = 2
NS = 16
NW = NC * NS
CHUNK = 128
D = 128


def _round_up(a, b):
    return (a + b - 1) // b * b


def _sc_mesh():
    return plsc.VectorSubcoreMesh(core_axis_name="c", subcore_axis_name="s")


def _degree_counts(dst2d, ones128, zeros128, n_pad):
    chunks = dst2d.shape[0]
    cps = chunks // NW
    rows_per_sub = n_pad // NS

    @functools.partial(
        pl.kernel,
        out_type=jax.ShapeDtypeStruct((NC, n_pad, D), jnp.float32),
        mesh=_sc_mesh(),
        scratch_types=[
            pltpu.VMEM((CHUNK,), jnp.int32),
            pltpu.VMEM((CHUNK, D), jnp.float32),
            pltpu.VMEM((CHUNK, D), jnp.float32),
            pltpu.VMEM_SHARED((n_pad, D), jnp.float32),
        ],
    )
    def deg_kernel(dst_hbm, ones_hbm, zeros_hbm, out_hbm, didx, ones_v, z16, deg_sh):
        c = lax.axis_index("c")
        s = lax.axis_index("s")
        w = c * NS + s
        pltpu.sync_copy(ones_hbm, ones_v)
        pltpu.sync_copy(zeros_hbm, z16)
        base = s * rows_per_sub

        @pl.loop(0, rows_per_sub, step=CHUNK)
        def _(r):
            pltpu.sync_copy(z16, deg_sh.at[pl.ds(base + r, CHUNK)])

        plsc.subcore_barrier()

        @pl.loop(0, cps)
        def _(j):
            pltpu.sync_copy(dst_hbm.at[w * cps + j], didx)
            pltpu.sync_copy(ones_v, deg_sh.at[didx], add=True)

        plsc.subcore_barrier()

        @pl.loop(0, rows_per_sub, step=CHUNK)
        def _(r):
            pltpu.sync_copy(deg_sh.at[pl.ds(base + r, CHUNK)],
                            out_hbm.at[c, pl.ds(base + r, CHUNK)])

    return deg_kernel(dst2d, ones128, zeros128)


def _scatter_rows(hprime, src2d, dst2d, zeros128, n_pad):
    chunks = src2d.shape[0]
    cps = chunks // NW
    rows_per_sub = n_pad // NS

    @functools.partial(
        pl.kernel,
        out_type=jax.ShapeDtypeStruct((NC, n_pad, D), jnp.float32),
        mesh=_sc_mesh(),
        scratch_types=[
            pltpu.VMEM((CHUNK,), jnp.int32),
            pltpu.VMEM((CHUNK,), jnp.int32),
            pltpu.VMEM((CHUNK, D), jnp.float32),
            pltpu.VMEM((CHUNK, D), jnp.float32),
            pltpu.VMEM_SHARED((n_pad, D), jnp.float32),
            pltpu.SemaphoreType.DMA,
        ],
    )
    def scat_kernel(h_hbm, src_hbm, dst_hbm, zeros_hbm, out_hbm,
                    sidx, didx, rows, zbuf, agg_sh, sem):
        c = lax.axis_index("c")
        s = lax.axis_index("s")
        w = c * NS + s
        pltpu.sync_copy(zeros_hbm, zbuf)
        base = s * rows_per_sub

        @pl.loop(0, rows_per_sub, step=CHUNK)
        def _(r):
            pltpu.sync_copy(zbuf, agg_sh.at[pl.ds(base + r, CHUNK)])

        plsc.subcore_barrier()

        @pl.loop(0, cps)
        def _(j):
            pltpu.sync_copy(src_hbm.at[w * cps + j], sidx)
            pltpu.sync_copy(dst_hbm.at[w * cps + j], didx)
            pltpu.async_copy(h_hbm.at[sidx], rows, sem).wait()
            pltpu.sync_copy(rows, agg_sh.at[didx], add=True)

        plsc.subcore_barrier()

        @pl.loop(0, rows_per_sub, step=CHUNK)
        def _(r):
            pltpu.sync_copy(agg_sh.at[pl.ds(base + r, CHUNK)],
                            out_hbm.at[c, pl.ds(base + r, CHUNK)])

    return scat_kernel(hprime, src2d, dst2d, zeros128)


def _tc_hprime(x_pad, W, deg):
    hp = x_pad.shape[0]

    def body(x_ref, w_ref, deg_ref, out_ref):
        h = jnp.dot(x_ref[...], w_ref[...], preferred_element_type=jnp.float32)
        d = deg_ref[0, :hp, 0:1] + deg_ref[1, :hp, 0:1] + 1.0
        out_ref[...] = h * lax.rsqrt(d)

    return pl.pallas_call(
        body, out_shape=jax.ShapeDtypeStruct((hp, D), jnp.float32)
    )(x_pad, W, deg)


def _tc_finish(agg, hprime, deg, b, n):
    def body(agg_ref, h_ref, deg_ref, b_ref, out_ref):
        acc = agg_ref[0, :n, :] + agg_ref[1, :n, :] + h_ref[:n, :]
        d = deg_ref[0, :n, 0:1] + deg_ref[1, :n, 0:1] + 1.0
        out_ref[...] = jnp.maximum(acc * lax.rsqrt(d) + b_ref[...], 0.0)

    return pl.pallas_call(
        body, out_shape=jax.ShapeDtypeStruct((n, D), jnp.float32)
    )(agg, hprime, deg, b.reshape(1, D))


def _prep_edges(edge_index, n):
    e = edge_index.shape[1]
    e_pad = _round_up(e, NW * CHUNK)
    pad = e_pad - e
    fill = jnp.full((pad,), n, jnp.int32)
    src = jnp.concatenate([edge_index[0], fill])
    dst = jnp.concatenate([edge_index[1], fill])
    return src.reshape(-1, CHUNK), dst.reshape(-1, CHUNK)


def _one_scale(x, edge_index, W, b, ones128, zeros128):
    n = x.shape[0]
    n_pad = _round_up(n + 1, NS * CHUNK)
    hp = _round_up(n + 1, 8)
    src2d, dst2d = _prep_edges(edge_index, n)
    deg = _degree_counts(dst2d, ones128, zeros128, n_pad)
    x_pad = jnp.pad(x, ((0, hp - n), (0, 0)))
    hprime = _tc_hprime(x_pad, W, deg)
    agg = _scatter_rows(hprime, src2d, dst2d, zeros128, n_pad)
    return _tc_finish(agg, hprime, deg, b, n)


def kernel(x0, x1, x2, edge_index0, edge_index1, edge_index2, W0, b0, W1, b1, W2, b2):
    ones128 = jnp.ones((CHUNK, D), jnp.float32)
    zeros128 = jnp.zeros((CHUNK, D), jnp.float32)
    out0 = _one_scale(x0, edge_index0, W0, b0, ones128, zeros128)
    out1 = _one_scale(x1, edge_index1, W1, b1, ones128, zeros128)
    out2 = _one_scale(x2, edge_index2, W2, b2, ones128, zeros128)
    return (out0, out1, out2)


# speedup vs baseline: 12.8103x; 1.0560x over previous
import functools

import jax
import jax.numpy as jnp
from jax import lax
from jax.experimental import pallas as pl
from jax.experimental.pallas import tpu as pltpu
from jax.experimental.pallas import tpu_sc as plsc

NC = 2
NS = 16
NW = NC * NS
CHUNK = 128
D = 128
NB = 2
SEG = 20
K_DEG = 8


def _round_up(a, b):
    return (a + b - 1) // b * b


def _sc_mesh():
    return plsc.VectorSubcoreMesh(core_axis_name="c", subcore_axis_name="s")


def _degree_counts(dst1d, ones128, zeros128, n_pad):
    chunks = dst1d.shape[0] // CHUNK
    cps = chunks // NW
    rows_per_sub = n_pad // NS

    @functools.partial(
        pl.kernel,
        out_type=jax.ShapeDtypeStruct((NC, n_pad, D), jnp.float32),
        mesh=_sc_mesh(),
        scratch_types=[
            pltpu.VMEM((cps * CHUNK,), jnp.int32),
            pltpu.VMEM((CHUNK, D), jnp.float32),
            pltpu.VMEM((CHUNK, D), jnp.float32),
            pltpu.VMEM_SHARED((n_pad, D), jnp.float32),
            pltpu.SemaphoreType.DMA,
        ],
    )
    def deg_kernel(dst_hbm, ones_hbm, zeros_hbm, out_hbm, didx_all, ones_v, z, deg_sh, dsem):
        c = lax.axis_index("c")
        s = lax.axis_index("s")
        w = c * NS + s
        pltpu.sync_copy(ones_hbm, ones_v)
        pltpu.sync_copy(zeros_hbm, z)
        base = s * rows_per_sub

        @pl.loop(0, rows_per_sub, step=CHUNK)
        def _(r):
            pltpu.sync_copy(z, deg_sh.at[pl.ds(base + r, CHUNK)])

        pltpu.sync_copy(dst_hbm.at[pl.ds(w * cps * CHUNK, cps * CHUNK)], didx_all)
        plsc.subcore_barrier()

        @pl.loop(0, cps)
        def _(j):
            @pl.when(j >= K_DEG)
            def _():
                pltpu.make_async_copy(
                    ones_v, deg_sh.at[didx_all.at[pl.ds(0, CHUNK)]], dsem).wait()

            pltpu.async_copy(
                ones_v, deg_sh.at[didx_all.at[pl.ds(j * CHUNK, CHUNK)]], dsem, add=True)

        @pl.loop(0, K_DEG)
        def _(j):
            pltpu.make_async_copy(
                ones_v, deg_sh.at[didx_all.at[pl.ds(0, CHUNK)]], dsem).wait()

        plsc.subcore_barrier()

        @pl.loop(0, rows_per_sub, step=CHUNK)
        def _(r):
            pltpu.sync_copy(deg_sh.at[pl.ds(base + r, CHUNK)],
                            out_hbm.at[c, pl.ds(base + r, CHUNK)])

    return deg_kernel(dst1d, ones128, zeros128)


def _scatter_rows(hprime, src1d, dst1d, zeros128, n_pad):
    chunks = src1d.shape[0] // CHUNK
    cps = chunks // NW
    rows_per_sub = n_pad // NS
    assert cps % SEG == 0 and SEG % NB == 0
    nseg = cps // SEG

    @functools.partial(
        pl.kernel,
        out_type=jax.ShapeDtypeStruct((NC, n_pad, D), jnp.float32),
        mesh=_sc_mesh(),
        scratch_types=[
            pltpu.VMEM((SEG * CHUNK,), jnp.int32),
            pltpu.VMEM((SEG * CHUNK,), jnp.int32),
            pltpu.VMEM((CHUNK, D), jnp.float32),
            pltpu.VMEM((CHUNK, D), jnp.float32),
            pltpu.VMEM_SHARED((n_pad, D), jnp.float32),
            pltpu.SemaphoreType.DMA,
            pltpu.SemaphoreType.DMA,
        ],
    )
    def scat_kernel(h_hbm, src_hbm, dst_hbm, zeros_hbm, out_hbm,
                    sidx_seg, didx_seg, r0, r1, agg_sh, gsem, ssem):
        rows = (r0, r1)
        c = lax.axis_index("c")
        s = lax.axis_index("s")
        w = c * NS + s
        base = s * rows_per_sub

        @pl.loop(0, rows_per_sub, step=CHUNK)
        def _(r):
            pltpu.sync_copy(zeros_hbm, agg_sh.at[pl.ds(base + r, CHUNK)])

        plsc.subcore_barrier()

        def sidx(j):
            return sidx_seg.at[pl.ds(j * CHUNK, CHUNK)]

        def didx(j):
            return didx_seg.at[pl.ds(j * CHUNK, CHUNK)]

        @pl.loop(0, nseg)
        def _(g):
            ebase = (w * cps + g * SEG) * CHUNK
            pltpu.sync_copy(src_hbm.at[pl.ds(ebase, SEG * CHUNK)], sidx_seg)
            pltpu.sync_copy(dst_hbm.at[pl.ds(ebase, SEG * CHUNK)], didx_seg)
            pltpu.async_copy(h_hbm.at[sidx(0)], rows[0], gsem)

            @pl.loop(0, SEG, step=NB)
            def _(j):
                for b in range(NB):
                    jj = j + b
                    b2 = (b + 1) % NB

                    @pl.when(jj + 1 < SEG)
                    def _():
                        @pl.when(jj + 1 >= NB)
                        def _():
                            pltpu.make_async_copy(
                                rows[b2], agg_sh.at[didx(0)], ssem).wait()

                        pltpu.async_copy(h_hbm.at[sidx(jj + 1)], rows[b2], gsem)

                    pltpu.make_async_copy(h_hbm.at[sidx(jj)], rows[b], gsem).wait()
                    pltpu.async_copy(rows[b], agg_sh.at[didx(jj)], ssem, add=True)

            for b in range(NB):
                pltpu.make_async_copy(rows[b], agg_sh.at[didx(0)], ssem).wait()

        plsc.subcore_barrier()

        @pl.loop(0, rows_per_sub, step=CHUNK)
        def _(r):
            pltpu.sync_copy(agg_sh.at[pl.ds(base + r, CHUNK)],
                            out_hbm.at[c, pl.ds(base + r, CHUNK)])

    return scat_kernel(hprime, src1d, dst1d, zeros128)


def _tc_hprime(x_pad, W, deg):
    hp = x_pad.shape[0]

    def body(x_ref, w_ref, deg_ref, out_ref):
        h = jnp.dot(x_ref[...], w_ref[...], preferred_element_type=jnp.float32)
        d = deg_ref[0, :hp, 0:1] + deg_ref[1, :hp, 0:1] + 1.0
        out_ref[...] = h * lax.rsqrt(d)

    return pl.pallas_call(
        body, out_shape=jax.ShapeDtypeStruct((hp, D), jnp.float32)
    )(x_pad, W, deg)


def _tc_finish(agg, hprime, deg, b, n):
    def body(agg_ref, h_ref, deg_ref, b_ref, out_ref):
        acc = agg_ref[0, :n, :] + agg_ref[1, :n, :] + h_ref[:n, :]
        d = deg_ref[0, :n, 0:1] + deg_ref[1, :n, 0:1] + 1.0
        out_ref[...] = jnp.maximum(acc * lax.rsqrt(d) + b_ref[...], 0.0)

    return pl.pallas_call(
        body, out_shape=jax.ShapeDtypeStruct((n, D), jnp.float32)
    )(agg, hprime, deg, b.reshape(1, D))


def _prep_edges(edge_index, n):
    e = edge_index.shape[1]
    e_pad = _round_up(e, NW * CHUNK * SEG)
    pad = e_pad - e
    fill = jnp.full((pad,), n, jnp.int32)
    src = jnp.concatenate([edge_index[0], fill])
    dst = jnp.concatenate([edge_index[1], fill])
    return src, dst


def _one_scale(x, edge_index, W, b, ones128, zeros128):
    n = x.shape[0]
    n_pad = _round_up(n + 1, NS * CHUNK)
    hp = _round_up(n + 1, 8)
    src2d, dst2d = _prep_edges(edge_index, n)
    deg = _degree_counts(dst2d, ones128, zeros128, n_pad)
    x_pad = jnp.pad(x, ((0, hp - n), (0, 0)))
    hprime = _tc_hprime(x_pad, W, deg)
    agg = _scatter_rows(hprime, src2d, dst2d, zeros128, n_pad)
    return _tc_finish(agg, hprime, deg, b, n)


def kernel(x0, x1, x2, edge_index0, edge_index1, edge_index2, W0, b0, W1, b1, W2, b2):
    ones128 = jnp.ones((CHUNK, D), jnp.float32)
    zeros128 = jnp.zeros((CHUNK, D), jnp.float32)
    out0 = _one_scale(x0, edge_index0, W0, b0, ones128, zeros128)
    out1 = _one_scale(x1, edge_index1, W1, b1, ones128, zeros128)
    out2 = _one_scale(x2, edge_index2, W2, b2, ones128, zeros128)
    return (out0, out1, out2)


# speedup vs baseline: 13.8437x; 1.0807x over previous
import functools

import jax
import jax.numpy as jnp
from jax import lax
from jax.experimental import pallas as pl
from jax.experimental.pallas import tpu as pltpu
from jax.experimental.pallas import tpu_sc as plsc

NC = 2
NS = 16
NW = NC * NS
CHUNK = 128
D = 128
NB = 2
SEG = 20
K_DEG = 8

SPMEM_BYTES = 9 * 512 * 1024


def _round_up(a, b):
    return (a + b - 1) // b * b


def _sc_mesh():
    return plsc.VectorSubcoreMesh(core_axis_name="c", subcore_axis_name="s")


def _degree_counts(dst1d, ones128, zeros128, n_pad):
    chunks = dst1d.shape[0] // CHUNK
    cps = chunks // NW
    rows_per_sub = n_pad // NS

    @functools.partial(
        pl.kernel,
        out_type=jax.ShapeDtypeStruct((NC, n_pad, D), jnp.float32),
        mesh=_sc_mesh(),
        scratch_types=[
            pltpu.VMEM((cps * CHUNK,), jnp.int32),
            pltpu.VMEM((CHUNK, D), jnp.float32),
            pltpu.VMEM((CHUNK, D), jnp.float32),
            pltpu.VMEM_SHARED((n_pad, D), jnp.float32),
            pltpu.SemaphoreType.DMA,
        ],
    )
    def deg_kernel(dst_hbm, ones_hbm, zeros_hbm, out_hbm, didx_all, ones_v, z, deg_sh, dsem):
        c = lax.axis_index("c")
        s = lax.axis_index("s")
        w = c * NS + s
        pltpu.sync_copy(ones_hbm, ones_v)
        pltpu.sync_copy(zeros_hbm, z)
        base = s * rows_per_sub

        @pl.loop(0, rows_per_sub, step=CHUNK)
        def _(r):
            pltpu.sync_copy(z, deg_sh.at[pl.ds(base + r, CHUNK)])

        pltpu.sync_copy(dst_hbm.at[pl.ds(w * cps * CHUNK, cps * CHUNK)], didx_all)
        plsc.subcore_barrier()

        @pl.loop(0, cps)
        def _(j):
            @pl.when(j >= K_DEG)
            def _():
                pltpu.make_async_copy(
                    ones_v, deg_sh.at[didx_all.at[pl.ds(0, CHUNK)]], dsem).wait()

            pltpu.async_copy(
                ones_v, deg_sh.at[didx_all.at[pl.ds(j * CHUNK, CHUNK)]], dsem, add=True)

        @pl.loop(0, K_DEG)
        def _(j):
            pltpu.make_async_copy(
                ones_v, deg_sh.at[didx_all.at[pl.ds(0, CHUNK)]], dsem).wait()

        plsc.subcore_barrier()

        @pl.loop(0, rows_per_sub, step=CHUNK)
        def _(r):
            pltpu.sync_copy(deg_sh.at[pl.ds(base + r, CHUNK)],
                            out_hbm.at[c, pl.ds(base + r, CHUNK)])

    return deg_kernel(dst1d, ones128, zeros128)


def _scatter_rows(hprime, src1d, dst1d, zeros128, n_pad, stage_h):
    chunks = src1d.shape[0] // CHUNK
    cps = chunks // NW
    rows_per_sub = n_pad // NS
    assert cps % SEG == 0 and SEG % NB == 0
    nseg = cps // SEG

    scratch = [
        pltpu.VMEM((SEG * CHUNK,), jnp.int32),
        pltpu.VMEM((SEG * CHUNK,), jnp.int32),
        pltpu.VMEM((CHUNK, D), jnp.float32),
        pltpu.VMEM((CHUNK, D), jnp.float32),
        pltpu.VMEM_SHARED((n_pad, D), jnp.float32),
        pltpu.SemaphoreType.DMA,
        pltpu.SemaphoreType.DMA,
    ]
    if stage_h:
        scratch.insert(4, pltpu.VMEM_SHARED((n_pad, D), jnp.float32))

    @functools.partial(
        pl.kernel,
        out_type=jax.ShapeDtypeStruct((NC, n_pad, D), jnp.float32),
        mesh=_sc_mesh(),
        scratch_types=scratch,
    )
    def scat_kernel(h_hbm, src_hbm, dst_hbm, zeros_hbm, out_hbm, *rest):
        if stage_h:
            sidx_seg, didx_seg, r0, r1, h_sh, agg_sh, gsem, ssem = rest
        else:
            sidx_seg, didx_seg, r0, r1, agg_sh, gsem, ssem = rest
            h_sh = None
        rows = (r0, r1)
        c = lax.axis_index("c")
        s = lax.axis_index("s")
        w = c * NS + s
        base = s * rows_per_sub

        @pl.loop(0, rows_per_sub, step=CHUNK)
        def _(r):
            pltpu.sync_copy(zeros_hbm, agg_sh.at[pl.ds(base + r, CHUNK)])
            if stage_h:
                pltpu.sync_copy(h_hbm.at[pl.ds(base + r, CHUNK)],
                                h_sh.at[pl.ds(base + r, CHUNK)])

        plsc.subcore_barrier()

        h_src = h_sh if stage_h else h_hbm

        def sidx(j):
            return sidx_seg.at[pl.ds(j * CHUNK, CHUNK)]

        def didx(j):
            return didx_seg.at[pl.ds(j * CHUNK, CHUNK)]

        @pl.loop(0, nseg)
        def _(g):
            ebase = (w * cps + g * SEG) * CHUNK
            pltpu.sync_copy(src_hbm.at[pl.ds(ebase, SEG * CHUNK)], sidx_seg)
            pltpu.sync_copy(dst_hbm.at[pl.ds(ebase, SEG * CHUNK)], didx_seg)
            pltpu.async_copy(h_src.at[sidx(0)], rows[0], gsem)

            @pl.loop(0, SEG, step=NB)
            def _(j):
                for b in range(NB):
                    jj = j + b
                    b2 = (b + 1) % NB

                    @pl.when(jj + 1 < SEG)
                    def _():
                        @pl.when(jj + 1 >= NB)
                        def _():
                            pltpu.make_async_copy(
                                rows[b2], agg_sh.at[didx(0)], ssem).wait()

                        pltpu.async_copy(h_src.at[sidx(jj + 1)], rows[b2], gsem)

                    pltpu.make_async_copy(h_src.at[sidx(jj)], rows[b], gsem).wait()
                    pltpu.async_copy(rows[b], agg_sh.at[didx(jj)], ssem, add=True)

            for b in range(NB):
                pltpu.make_async_copy(rows[b], agg_sh.at[didx(0)], ssem).wait()

        plsc.subcore_barrier()

        @pl.loop(0, rows_per_sub, step=CHUNK)
        def _(r):
            pltpu.sync_copy(agg_sh.at[pl.ds(base + r, CHUNK)],
                            out_hbm.at[c, pl.ds(base + r, CHUNK)])

    return scat_kernel(hprime, src1d, dst1d, zeros128)


def _tc_hprime(x_pad, W, deg):
    hp = x_pad.shape[0]

    def body(x_ref, w_ref, deg_ref, out_ref):
        h = jnp.dot(x_ref[...], w_ref[...], preferred_element_type=jnp.float32)
        d = deg_ref[0, :hp, 0:1] + deg_ref[1, :hp, 0:1] + 1.0
        out_ref[...] = h * lax.rsqrt(d)

    return pl.pallas_call(
        body, out_shape=jax.ShapeDtypeStruct((hp, D), jnp.float32)
    )(x_pad, W, deg)


def _tc_finish(agg, hprime, deg, b, n):
    def body(agg_ref, h_ref, deg_ref, b_ref, out_ref):
        acc = agg_ref[0, :n, :] + agg_ref[1, :n, :] + h_ref[:n, :]
        d = deg_ref[0, :n, 0:1] + deg_ref[1, :n, 0:1] + 1.0
        out_ref[...] = jnp.maximum(acc * lax.rsqrt(d) + b_ref[...], 0.0)

    return pl.pallas_call(
        body, out_shape=jax.ShapeDtypeStruct((n, D), jnp.float32)
    )(agg, hprime, deg, b.reshape(1, D))


def _prep_edges(edge_index, n):
    e = edge_index.shape[1]
    e_pad = _round_up(e, NW * CHUNK * SEG)
    pad = e_pad - e
    fill = jnp.full((pad,), n, jnp.int32)
    src = jnp.concatenate([edge_index[0], fill])
    dst = jnp.concatenate([edge_index[1], fill])
    return src, dst


def _one_scale(x, edge_index, W, b, ones128, zeros128):
    n = x.shape[0]
    n_pad = _round_up(n + 1, NS * CHUNK)
    src1d, dst1d = _prep_edges(edge_index, n)
    deg = _degree_counts(dst1d, ones128, zeros128, n_pad)
    stage_h = 2 * n_pad * D * 4 <= SPMEM_BYTES
    hp = n_pad if stage_h else _round_up(n + 1, 8)
    x_pad = jnp.pad(x, ((0, hp - n), (0, 0)))
    hprime = _tc_hprime(x_pad, W, deg)
    agg = _scatter_rows(hprime, src1d, dst1d, zeros128, n_pad, stage_h)
    return _tc_finish(agg, hprime, deg, b, n)


def kernel(x0, x1, x2, edge_index0, edge_index1, edge_index2, W0, b0, W1, b1, W2, b2):
    ones128 = jnp.ones((CHUNK, D), jnp.float32)
    zeros128 = jnp.zeros((CHUNK, D), jnp.float32)
    out0 = _one_scale(x0, edge_index0, W0, b0, ones128, zeros128)
    out1 = _one_scale(x1, edge_index1, W1, b1, ones128, zeros128)
    out2 = _one_scale(x2, edge_index2, W2, b2, ones128, zeros128)
    return (out0, out1, out2)


# speedup vs baseline: 18.3249x; 1.3237x over previous
import functools

import jax
import jax.numpy as jnp
from jax import lax
from jax.experimental import pallas as pl
from jax.experimental.pallas import tpu as pltpu
from jax.experimental.pallas import tpu_sc as plsc

NC = 2
NS = 16
NW = NC * NS
CHUNK = 128
D = 128
NB = 2
SEG = 20
K_DEG = 8

SPMEM_BYTES = 5_800_000


def _round_up(a, b):
    return (a + b - 1) // b * b


def _sc_mesh():
    return plsc.VectorSubcoreMesh(core_axis_name="c", subcore_axis_name="s")


def _degree_counts(dst1d, ones128, zeros128, n_pad):
    chunks = dst1d.shape[0] // CHUNK
    cps = chunks // NW
    rows_per_sub = n_pad // NS

    @functools.partial(
        pl.kernel,
        out_type=jax.ShapeDtypeStruct((NC, n_pad, D), jnp.float32),
        mesh=_sc_mesh(),
        scratch_types=[
            pltpu.VMEM((cps * CHUNK,), jnp.int32),
            pltpu.VMEM((CHUNK, D), jnp.float32),
            pltpu.VMEM((CHUNK, D), jnp.float32),
            pltpu.VMEM_SHARED((n_pad, D), jnp.float32),
            pltpu.SemaphoreType.DMA,
        ],
    )
    def deg_kernel(dst_hbm, ones_hbm, zeros_hbm, out_hbm, didx_all, ones_v, z, deg_sh, dsem):
        c = lax.axis_index("c")
        s = lax.axis_index("s")
        w = c * NS + s
        pltpu.sync_copy(ones_hbm, ones_v)
        pltpu.sync_copy(zeros_hbm, z)
        base = s * rows_per_sub

        @pl.loop(0, rows_per_sub, step=CHUNK)
        def _(r):
            pltpu.sync_copy(z, deg_sh.at[pl.ds(base + r, CHUNK)])

        pltpu.sync_copy(dst_hbm.at[pl.ds(w * cps * CHUNK, cps * CHUNK)], didx_all)
        plsc.subcore_barrier()

        @pl.loop(0, cps)
        def _(j):
            @pl.when(j >= K_DEG)
            def _():
                pltpu.make_async_copy(
                    ones_v, deg_sh.at[didx_all.at[pl.ds(0, CHUNK)]], dsem).wait()

            pltpu.async_copy(
                ones_v, deg_sh.at[didx_all.at[pl.ds(j * CHUNK, CHUNK)]], dsem, add=True)

        @pl.loop(0, K_DEG)
        def _(j):
            pltpu.make_async_copy(
                ones_v, deg_sh.at[didx_all.at[pl.ds(0, CHUNK)]], dsem).wait()

        plsc.subcore_barrier()

        @pl.loop(0, rows_per_sub, step=CHUNK)
        def _(r):
            pltpu.sync_copy(deg_sh.at[pl.ds(base + r, CHUNK)],
                            out_hbm.at[c, pl.ds(base + r, CHUNK)])

    return deg_kernel(dst1d, ones128, zeros128)


def _scatter_rows(hprime, src1d, dst1d, zeros128, n_pad, stage_h):
    chunks = src1d.shape[0] // CHUNK
    cps = chunks // NW
    assert n_pad % CHUNK == 0 and cps % SEG == 0 and SEG % NB == 0
    nseg = cps // SEG
    kmax = (n_pad // CHUNK + NS - 1) // NS

    sh_rows = 2 * n_pad if stage_h else n_pad
    agg0 = n_pad if stage_h else 0

    @functools.partial(
        pl.kernel,
        out_type=jax.ShapeDtypeStruct((NC, n_pad, D), jnp.float32),
        mesh=_sc_mesh(),
        scratch_types=[
            pltpu.VMEM((SEG * CHUNK,), jnp.int32),
            pltpu.VMEM((SEG * CHUNK,), jnp.int32),
            pltpu.VMEM((CHUNK, D), jnp.float32),
            pltpu.VMEM((CHUNK, D), jnp.float32),
            pltpu.VMEM_SHARED((sh_rows, D), jnp.float32),
            pltpu.SemaphoreType.DMA,
            pltpu.SemaphoreType.DMA,
        ],
    )
    def scat_kernel(h_hbm, src_hbm, dst_hbm, zeros_hbm, out_hbm,
                    sidx_seg, didx_seg, r0, r1, comb_sh, gsem, ssem):
        rows = (r0, r1)
        c = lax.axis_index("c")
        s = lax.axis_index("s")
        w = c * NS + s

        @pl.loop(0, kmax)
        def _(k):
            r = (s + k * NS) * CHUNK

            @pl.when(r < n_pad)
            def _():
                pltpu.sync_copy(zeros_hbm, comb_sh.at[pl.ds(agg0 + r, CHUNK)])
                if stage_h:
                    pltpu.sync_copy(h_hbm.at[pl.ds(r, CHUNK)],
                                    comb_sh.at[pl.ds(r, CHUNK)])

        plsc.subcore_barrier()

        h_src = comb_sh if stage_h else h_hbm
        agg_sh = comb_sh

        def sidx(j):
            return sidx_seg.at[pl.ds(j * CHUNK, CHUNK)]

        def didx(j):
            return didx_seg.at[pl.ds(j * CHUNK, CHUNK)]

        @pl.loop(0, nseg)
        def _(g):
            ebase = (w * cps + g * SEG) * CHUNK
            pltpu.sync_copy(src_hbm.at[pl.ds(ebase, SEG * CHUNK)], sidx_seg)
            pltpu.sync_copy(dst_hbm.at[pl.ds(ebase, SEG * CHUNK)], didx_seg)
            pltpu.async_copy(h_src.at[sidx(0)], rows[0], gsem)

            @pl.loop(0, SEG, step=NB)
            def _(j):
                for b in range(NB):
                    jj = j + b
                    b2 = (b + 1) % NB

                    @pl.when(jj + 1 < SEG)
                    def _():
                        @pl.when(jj + 1 >= NB)
                        def _():
                            pltpu.make_async_copy(
                                rows[b2], agg_sh.at[didx(0)], ssem).wait()

                        pltpu.async_copy(h_src.at[sidx(jj + 1)], rows[b2], gsem)

                    pltpu.make_async_copy(h_src.at[sidx(jj)], rows[b], gsem).wait()
                    pltpu.async_copy(rows[b], agg_sh.at[didx(jj)], ssem, add=True)

            for b in range(NB):
                pltpu.make_async_copy(rows[b], agg_sh.at[didx(0)], ssem).wait()

        plsc.subcore_barrier()

        @pl.loop(0, kmax)
        def _(k):
            r = (s + k * NS) * CHUNK

            @pl.when(r < n_pad)
            def _():
                pltpu.sync_copy(agg_sh.at[pl.ds(agg0 + r, CHUNK)],
                                out_hbm.at[c, pl.ds(r, CHUNK)])

    return scat_kernel(hprime, src1d, dst1d, zeros128)


def _tc_hprime(x_pad, W, deg):
    hp = x_pad.shape[0]

    def body(x_ref, w_ref, deg_ref, out_ref):
        h = jnp.dot(x_ref[...], w_ref[...], preferred_element_type=jnp.float32)
        d = deg_ref[0, :hp, 0:1] + deg_ref[1, :hp, 0:1] + 1.0
        out_ref[...] = h * lax.rsqrt(d)

    return pl.pallas_call(
        body, out_shape=jax.ShapeDtypeStruct((hp, D), jnp.float32)
    )(x_pad, W, deg)


def _tc_finish(agg, hprime, deg, b, n):
    def body(agg_ref, h_ref, deg_ref, b_ref, out_ref):
        acc = agg_ref[0, :n, :] + agg_ref[1, :n, :] + h_ref[:n, :]
        d = deg_ref[0, :n, 0:1] + deg_ref[1, :n, 0:1] + 1.0
        out_ref[...] = jnp.maximum(acc * lax.rsqrt(d) + b_ref[...], 0.0)

    return pl.pallas_call(
        body, out_shape=jax.ShapeDtypeStruct((n, D), jnp.float32)
    )(agg, hprime, deg, b.reshape(1, D))


def _prep_edges(edge_index, n):
    e = edge_index.shape[1]
    e_pad = _round_up(e, NW * CHUNK * SEG)
    pad = e_pad - e
    fill = jnp.full((pad,), n, jnp.int32)
    src = jnp.concatenate([edge_index[0], fill])
    dst = jnp.concatenate([edge_index[1], fill])
    return src, dst


def _one_scale(x, edge_index, W, b, ones128, zeros128):
    n = x.shape[0]
    n_deg = _round_up(n + 1, NS * CHUNK)
    n_pad = _round_up(n + 1, CHUNK)
    src1d, dst1d = _prep_edges(edge_index, n)
    deg = _degree_counts(dst1d, ones128, zeros128, n_deg)
    stage_h = 2 * n_pad * D * 4 <= SPMEM_BYTES
    hp = n_pad if stage_h else _round_up(n + 1, 8)
    x_pad = jnp.pad(x, ((0, hp - n), (0, 0)))
    hprime = _tc_hprime(x_pad, W, deg)
    if stage_h:
        dst1d = dst1d + n_pad
    agg = _scatter_rows(hprime, src1d, dst1d, zeros128, n_pad, stage_h)
    return _tc_finish(agg, hprime, deg, b, n)


def kernel(x0, x1, x2, edge_index0, edge_index1, edge_index2, W0, b0, W1, b1, W2, b2):
    ones128 = jnp.ones((CHUNK, D), jnp.float32)
    zeros128 = jnp.zeros((CHUNK, D), jnp.float32)
    out0 = _one_scale(x0, edge_index0, W0, b0, ones128, zeros128)
    out1 = _one_scale(x1, edge_index1, W1, b1, ones128, zeros128)
    out2 = _one_scale(x2, edge_index2, W2, b2, ones128, zeros128)
    return (out0, out1, out2)
